# baseline (speedup 1.0000x reference)
import functools

import jax
import jax.numpy as jnp
from jax import lax
from jax.experimental import pallas as pl
from jax.experimental.pallas import tpu as pltpu
from jax.experimental.pallas import tpu_sc as plsc

N = 8192
L = 64
H = 2048
IN = 2 * L
B = 640
NPAD = ((N + B - 1) // B + 1) * B
NBP = NPAD // B
NQ = 12

NC, NS = 2, 16
NW = NC * NS


def _make_row_gather(n_rows, n_cols, dtype, chunk):
    per_w = n_rows // NW
    assert per_w % chunk == 0 and chunk % 8 == 0 and chunk <= 128
    n_chunks = per_w // chunk
    mesh = plsc.VectorSubcoreMesh(core_axis_name="c", subcore_axis_name="s")

    def body(table_hbm, idx_hbm, out_hbm, idx_v, rows_v, sem):
        wid = lax.axis_index("s") * NC + lax.axis_index("c")
        base = wid * per_w
        for j in range(n_chunks):
            off = base + j * chunk
            pltpu.sync_copy(idx_hbm.at[pl.ds(off, chunk)], idx_v)
            pltpu.async_copy(table_hbm.at[idx_v], rows_v, sem).wait()
            pltpu.sync_copy(rows_v, out_hbm.at[pl.ds(off, chunk)])

    return pl.kernel(
        body,
        mesh=mesh,
        out_type=jax.ShapeDtypeStruct((n_rows, n_cols), dtype),
        scratch_types=[
            pltpu.VMEM((chunk,), jnp.int32),
            pltpu.VMEM((chunk, n_cols), dtype),
            pltpu.SemaphoreType.DMA,
        ],
    )


def _embed(q, parity, ea, et, dim):
    acc = jnp.zeros(q.shape, jnp.float32)
    for k in range(NQ):
        val = jnp.where(parity > 0, et[k, dim], ea[k, dim])
        acc = acc + jnp.where(q == k, val, 0.0)
    return acc.astype(jnp.bfloat16)


C1 = 1024
C2 = 1024


def _expert_mlp(x, W1_ref, b1_ref, W2_ref, b2_ref, h_ref, out_ref):
    dot = functools.partial(jnp.dot, preferred_element_type=jnp.float32)
    for c in range(0, 2 * H, C1):
        hc = dot(x, W1_ref[:, c:c + C1]) + b1_ref[:, c:c + C1]
        h_ref[:, c:c + C1] = jnp.maximum(hc, 0.0).astype(jnp.bfloat16)
    h = h_ref[...]
    for d in range(0, H, C2):
        out_ref[:, d:d + C2] = dot(h, W2_ref[:, d:d + C2]) + b2_ref[:, d:d + C2]


def _mlp_body(eid_ref, sig_ref, eac, etc, eal, etl,
              W1c_ref, b1c_ref, W2c_ref, b2c_ref,
              W1l_ref, b1l_ref, W2l_ref, b2l_ref, out_ref, h_ref):
    s = sig_ref[...]
    q = jnp.where(s < 1e-8,
                  jnp.zeros(s.shape, jnp.int32),
                  jnp.floor(s * 10.0).astype(jnp.int32) + 1)
    parity = lax.broadcasted_iota(jnp.int32, (1, IN), 1) % 2
    e = eid_ref[pl.program_id(0)]

    ea = jnp.where(e > 0, eal[...], eac[...])
    et = jnp.where(e > 0, etl[...], etc[...])
    x0 = _embed(q, parity, ea, et, 0)
    x1 = _embed(q, parity, ea, et, 1)
    x = jnp.concatenate([x0, x1], axis=1)

    @pl.when(e == 0)
    def _():
        _expert_mlp(x, W1c_ref, b1c_ref, W2c_ref, b2c_ref, h_ref, out_ref)

    @pl.when(e != 0)
    def _():
        _expert_mlp(x, W1l_ref, b1l_ref, W2l_ref, b2l_ref, h_ref, out_ref)


def _w1_split(W1):
    W = W1.reshape(L, 2, 2, 2 * H)
    WA = W[:, :, 0, :].reshape(IN, 2 * H)
    WB = W[:, :, 1, :].reshape(IN, 2 * H)
    return jnp.concatenate([WA, WB]).astype(jnp.bfloat16)


def kernel(signatures, selector, emb_app_c, emb_tf_c, emb_app_l, emb_tf_l,
           W1c, b1c, W2c, b2c, W1l, b1l, W2l, b2l):
    is_c = (selector == 0).astype(jnp.int32)
    r0 = jnp.cumsum(is_c) - is_c
    r1 = jnp.cumsum(1 - is_c) - (1 - is_c)
    n0 = jnp.sum(is_c)
    n0p = ((n0 + B - 1) // B) * B
    row = jnp.where(is_c > 0, r0, n0p + r1)
    src = jnp.zeros((NPAD,), jnp.int32).at[row].set(
        jnp.arange(N, dtype=jnp.int32))
    eid = (jnp.arange(NBP, dtype=jnp.int32) * B >= n0p).astype(jnp.int32)

    sig2 = signatures.reshape(N, IN)
    sig_sorted = _make_row_gather(NPAD, IN, jnp.float32, chunk=56)(sig2, src)

    W1cs, W1ls = _w1_split(W1c), _w1_split(W1l)
    W2cb = W2c.astype(jnp.bfloat16)
    W2lb = W2l.astype(jnp.bfloat16)
    b1c2 = b1c.reshape(1, 2 * H)
    b1l2 = b1l.reshape(1, 2 * H)
    b2c2 = b2c.reshape(1, H)
    b2l2 = b2l.reshape(1, H)

    full = lambda shape: pl.BlockSpec(shape, lambda it, eid: (0,) * len(shape))
    grid_spec = pltpu.PrefetchScalarGridSpec(
        num_scalar_prefetch=1,
        grid=(NBP,),
        in_specs=[
            pl.BlockSpec((B, IN), lambda it, eid: (it, 0)),
            full((NQ, 2)), full((NQ, 2)), full((NQ, 2)), full((NQ, 2)),
            full((2 * IN, 2 * H)), full((1, 2 * H)),
            full((2 * H, H)), full((1, H)),
            full((2 * IN, 2 * H)), full((1, 2 * H)),
            full((2 * H, H)), full((1, H)),
        ],
        out_specs=pl.BlockSpec((B, H), lambda it, eid: (it, 0)),
        scratch_shapes=[pltpu.VMEM((B, 2 * H), jnp.bfloat16)],
    )
    y_sorted = pl.pallas_call(
        _mlp_body,
        grid_spec=grid_spec,
        out_shape=jax.ShapeDtypeStruct((NPAD, H), jnp.float32),
    )(eid, sig_sorted, emb_app_c, emb_tf_c, emb_app_l, emb_tf_l,
      W1cs, b1c2, W2cb, b2c2, W1ls, b1l2, W2lb, b2l2)

    return _make_row_gather(N, H, jnp.float32, chunk=32)(y_sorted, row)


# speedup vs baseline: 1.0078x; 1.0078x over previous
import functools

import jax
import jax.numpy as jnp
from jax import lax
from jax.experimental import pallas as pl
from jax.experimental.pallas import tpu as pltpu
from jax.experimental.pallas import tpu_sc as plsc

N = 8192
L = 64
H = 2048
IN = 2 * L
B = 640
NPAD = ((N + B - 1) // B + 1) * B
NBP = NPAD // B
NQ = 12

NC, NS = 2, 16
NW = NC * NS


def _make_row_gather(n_rows, n_cols, dtype, chunks):
    per_w = n_rows // NW
    assert sum(chunks) == per_w
    assert all(c % 8 == 0 and c <= 128 for c in chunks)
    cmax = max(chunks)
    offs = [sum(chunks[:j]) for j in range(len(chunks))]
    mesh = plsc.VectorSubcoreMesh(core_axis_name="c", subcore_axis_name="s")

    def body(table_hbm, idx_hbm, out_hbm, idx_all, rows0, rows1, gsem, osem0,
             osem1):
        wid = lax.axis_index("s") * NC + lax.axis_index("c")
        base = wid * per_w
        pltpu.sync_copy(idx_hbm.at[pl.ds(base, per_w)], idx_all)
        rows = (rows0, rows1)
        osems = (osem0, osem1)
        scat = [None, None]
        for j, (off, c) in enumerate(zip(offs, chunks)):
            b = j & 1
            if scat[b] is not None:
                scat[b].wait()
            pltpu.async_copy(
                table_hbm.at[idx_all.at[pl.ds(off, c)]],
                rows[b].at[pl.ds(0, c)], gsem).wait()
            scat[b] = pltpu.async_copy(
                rows[b].at[pl.ds(0, c)],
                out_hbm.at[pl.ds(base + off, c)], osems[b])
        for b in (0, 1):
            if scat[b] is not None:
                scat[b].wait()

    return pl.kernel(
        body,
        mesh=mesh,
        out_type=jax.ShapeDtypeStruct((n_rows, n_cols), dtype),
        scratch_types=[
            pltpu.VMEM((per_w,), jnp.int32),
            pltpu.VMEM((cmax, n_cols), dtype),
            pltpu.VMEM((cmax, n_cols), dtype),
            pltpu.SemaphoreType.DMA,
            pltpu.SemaphoreType.DMA,
            pltpu.SemaphoreType.DMA,
        ],
    )


def _embed(q, parity, ea, et, dim):
    acc = jnp.zeros(q.shape, jnp.float32)
    for k in range(NQ):
        val = jnp.where(parity > 0, et[k, dim], ea[k, dim])
        acc = acc + jnp.where(q == k, val, 0.0)
    return acc.astype(jnp.bfloat16)


C1 = 1024
C2 = 1024


def _expert_mlp(x, W1_ref, b1_ref, W2_ref, b2_ref, h_ref, out_ref):
    dot = functools.partial(jnp.dot, preferred_element_type=jnp.float32)
    for c in range(0, 2 * H, C1):
        hc = dot(x, W1_ref[:, c:c + C1]) + b1_ref[:, c:c + C1]
        h_ref[:, c:c + C1] = jnp.maximum(hc, 0.0).astype(jnp.bfloat16)
    h = h_ref[...]
    for d in range(0, H, C2):
        out_ref[:, d:d + C2] = dot(h, W2_ref[:, d:d + C2]) + b2_ref[:, d:d + C2]


def _mlp_body(eid_ref, sig_ref, eac, etc, eal, etl,
              W1c_ref, b1c_ref, W2c_ref, b2c_ref,
              W1l_ref, b1l_ref, W2l_ref, b2l_ref, out_ref, h_ref):
    s = sig_ref[...]
    q = jnp.where(s < 1e-8,
                  jnp.zeros(s.shape, jnp.int32),
                  jnp.floor(s * 10.0).astype(jnp.int32) + 1)
    parity = lax.broadcasted_iota(jnp.int32, (1, IN), 1) % 2
    e = eid_ref[pl.program_id(0)]

    ea = jnp.where(e > 0, eal[...], eac[...])
    et = jnp.where(e > 0, etl[...], etc[...])
    x0 = _embed(q, parity, ea, et, 0)
    x1 = _embed(q, parity, ea, et, 1)
    x = jnp.concatenate([x0, x1], axis=1)

    @pl.when(e == 0)
    def _():
        _expert_mlp(x, W1c_ref, b1c_ref, W2c_ref, b2c_ref, h_ref, out_ref)

    @pl.when(e != 0)
    def _():
        _expert_mlp(x, W1l_ref, b1l_ref, W2l_ref, b2l_ref, h_ref, out_ref)


def _w1_split(W1):
    W = W1.reshape(L, 2, 2, 2 * H)
    WA = W[:, :, 0, :].reshape(IN, 2 * H)
    WB = W[:, :, 1, :].reshape(IN, 2 * H)
    return jnp.concatenate([WA, WB]).astype(jnp.bfloat16)


def kernel(signatures, selector, emb_app_c, emb_tf_c, emb_app_l, emb_tf_l,
           W1c, b1c, W2c, b2c, W1l, b1l, W2l, b2l):
    is_c = (selector == 0).astype(jnp.int32)
    cs = jnp.cumsum(is_c)
    r0 = cs - is_c
    r1 = jnp.arange(N, dtype=jnp.int32) - r0
    n0 = cs[N - 1]
    n0p = ((n0 + B - 1) // B) * B
    row = jnp.where(is_c > 0, r0, n0p + r1)
    src = jnp.zeros((NPAD,), jnp.int32).at[row].set(
        jnp.arange(N, dtype=jnp.int32))
    eid = (jnp.arange(NBP, dtype=jnp.int32) * B >= n0p).astype(jnp.int32)

    sig2 = signatures.reshape(N, IN)
    sig_sorted = _make_row_gather(NPAD, IN, jnp.float32, chunks=[128, 128, 24])(sig2, src)

    W1cs, W1ls = _w1_split(W1c), _w1_split(W1l)
    W2cb = W2c.astype(jnp.bfloat16)
    W2lb = W2l.astype(jnp.bfloat16)
    b1c2 = b1c.reshape(1, 2 * H)
    b1l2 = b1l.reshape(1, 2 * H)
    b2c2 = b2c.reshape(1, H)
    b2l2 = b2l.reshape(1, H)

    full = lambda shape: pl.BlockSpec(shape, lambda it, eid: (0,) * len(shape))
    grid_spec = pltpu.PrefetchScalarGridSpec(
        num_scalar_prefetch=1,
        grid=(NBP,),
        in_specs=[
            pl.BlockSpec((B, IN), lambda it, eid: (it, 0)),
            full((NQ, 2)), full((NQ, 2)), full((NQ, 2)), full((NQ, 2)),
            full((2 * IN, 2 * H)), full((1, 2 * H)),
            full((2 * H, H)), full((1, H)),
            full((2 * IN, 2 * H)), full((1, 2 * H)),
            full((2 * H, H)), full((1, H)),
        ],
        out_specs=pl.BlockSpec((B, H), lambda it, eid: (it, 0)),
        scratch_shapes=[pltpu.VMEM((B, 2 * H), jnp.bfloat16)],
    )
    y_sorted = pl.pallas_call(
        _mlp_body,
        grid_spec=grid_spec,
        out_shape=jax.ShapeDtypeStruct((NPAD, H), jnp.float32),
    )(eid, sig_sorted, emb_app_c, emb_tf_c, emb_app_l, emb_tf_l,
      W1cs, b1c2, W2cb, b2c2, W1ls, b1l2, W2lb, b2l2)

    return _make_row_gather(N, H, jnp.float32, chunks=[24] * 10 + [16])(y_sorted, row)


# speedup vs baseline: 1.0657x; 1.0574x over previous
import functools

import jax
import jax.numpy as jnp
from jax import lax
from jax.experimental import pallas as pl
from jax.experimental.pallas import tpu as pltpu
from jax.experimental.pallas import tpu_sc as plsc

N = 8192
L = 64
H = 2048
IN = 2 * L
B = 512
NPAD = ((N + B - 1) // B + 1) * B
NBP = NPAD // B
NQ = 12

NC, NS = 2, 16
NW = NC * NS


def _make_row_gather(n_rows, n_cols, dtype, chunks):
    per_w = n_rows // NW
    assert sum(chunks) == per_w
    assert all(c % 8 == 0 and c <= 128 for c in chunks)
    cmax = max(chunks)
    offs = [sum(chunks[:j]) for j in range(len(chunks))]
    mesh = plsc.VectorSubcoreMesh(core_axis_name="c", subcore_axis_name="s")

    def body(table_hbm, idx_hbm, out_hbm, idx_all, rows0, rows1, gsem, osem0,
             osem1):
        wid = lax.axis_index("s") * NC + lax.axis_index("c")
        base = wid * per_w
        pltpu.sync_copy(idx_hbm.at[pl.ds(base, per_w)], idx_all)
        rows = (rows0, rows1)
        osems = (osem0, osem1)
        scat = [None, None]
        for j, (off, c) in enumerate(zip(offs, chunks)):
            b = j & 1
            if scat[b] is not None:
                scat[b].wait()
            pltpu.async_copy(
                table_hbm.at[idx_all.at[pl.ds(off, c)]],
                rows[b].at[pl.ds(0, c)], gsem).wait()
            scat[b] = pltpu.async_copy(
                rows[b].at[pl.ds(0, c)],
                out_hbm.at[pl.ds(base + off, c)], osems[b])
        for b in (0, 1):
            if scat[b] is not None:
                scat[b].wait()

    return pl.kernel(
        body,
        mesh=mesh,
        out_type=jax.ShapeDtypeStruct((n_rows, n_cols), dtype),
        scratch_types=[
            pltpu.VMEM((per_w,), jnp.int32),
            pltpu.VMEM((cmax, n_cols), dtype),
            pltpu.VMEM((cmax, n_cols), dtype),
            pltpu.SemaphoreType.DMA,
            pltpu.SemaphoreType.DMA,
            pltpu.SemaphoreType.DMA,
        ],
    )


def _embed(q, parity, ea, et, dim):
    acc = jnp.zeros(q.shape, jnp.float32)
    for k in range(NQ):
        val = jnp.where(parity > 0, et[k, dim], ea[k, dim])
        acc = acc + jnp.where(q == k, val, 0.0)
    return acc.astype(jnp.bfloat16)


C1 = 1024
C2 = 1024


def _expert_mlp(x, W1_ref, b1_ref, W2_ref, b2_ref, h_ref, out_ref):
    dot = functools.partial(jnp.dot, preferred_element_type=jnp.float32)
    for c in range(0, 2 * H, C1):
        hc = dot(x, W1_ref[:, c:c + C1]) + b1_ref[:, c:c + C1]
        h_ref[:, c:c + C1] = jnp.maximum(hc, 0.0).astype(jnp.bfloat16)
    h = h_ref[...]
    for d in range(0, H, C2):
        out_ref[:, d:d + C2] = dot(h, W2_ref[:, d:d + C2]) + b2_ref[:, d:d + C2]


def _mlp_body(eid_ref, sig_ref, eac, etc, eal, etl,
              W1c_ref, b1c_ref, W2c_ref, b2c_ref,
              W1l_ref, b1l_ref, W2l_ref, b2l_ref, out_ref, h_ref):
    s = sig_ref[...]
    q = jnp.where(s < 1e-8,
                  jnp.zeros(s.shape, jnp.int32),
                  jnp.floor(s * 10.0).astype(jnp.int32) + 1)
    parity = lax.broadcasted_iota(jnp.int32, (1, IN), 1) % 2
    e = eid_ref[pl.program_id(0)]

    ea = jnp.where(e > 0, eal[...], eac[...])
    et = jnp.where(e > 0, etl[...], etc[...])
    x0 = _embed(q, parity, ea, et, 0)
    x1 = _embed(q, parity, ea, et, 1)
    x = jnp.concatenate([x0, x1], axis=1)

    @pl.when(e == 0)
    def _():
        _expert_mlp(x, W1c_ref, b1c_ref, W2c_ref, b2c_ref, h_ref, out_ref)

    @pl.when(e != 0)
    def _():
        _expert_mlp(x, W1l_ref, b1l_ref, W2l_ref, b2l_ref, h_ref, out_ref)


def _w1_split(W1):
    W = W1.reshape(L, 2, 2, 2 * H)
    WA = W[:, :, 0, :].reshape(IN, 2 * H)
    WB = W[:, :, 1, :].reshape(IN, 2 * H)
    return jnp.concatenate([WA, WB]).astype(jnp.bfloat16)


def kernel(signatures, selector, emb_app_c, emb_tf_c, emb_app_l, emb_tf_l,
           W1c, b1c, W2c, b2c, W1l, b1l, W2l, b2l):
    is_c = (selector == 0).astype(jnp.int32)
    cs = jnp.cumsum(is_c)
    r0 = cs - is_c
    r1 = jnp.arange(N, dtype=jnp.int32) - r0
    n0 = cs[N - 1]
    n0p = ((n0 + B - 1) // B) * B
    row = jnp.where(is_c > 0, r0, n0p + r1)
    src = jnp.zeros((NPAD,), jnp.int32).at[row].set(
        jnp.arange(N, dtype=jnp.int32))
    eid = (jnp.arange(NBP, dtype=jnp.int32) * B >= n0p).astype(jnp.int32)

    sig2 = signatures.reshape(N, IN)
    sig_sorted = _make_row_gather(NPAD, IN, jnp.float32, chunks=[128, 128, 16])(sig2, src)

    W1cs, W1ls = _w1_split(W1c), _w1_split(W1l)
    W2cb = W2c.astype(jnp.bfloat16)
    W2lb = W2l.astype(jnp.bfloat16)
    b1c2 = b1c.reshape(1, 2 * H)
    b1l2 = b1l.reshape(1, 2 * H)
    b2c2 = b2c.reshape(1, H)
    b2l2 = b2l.reshape(1, H)

    full = lambda shape: pl.BlockSpec(shape, lambda it, eid: (0,) * len(shape))
    grid_spec = pltpu.PrefetchScalarGridSpec(
        num_scalar_prefetch=1,
        grid=(NBP,),
        in_specs=[
            pl.BlockSpec((B, IN), lambda it, eid: (it, 0)),
            full((NQ, 2)), full((NQ, 2)), full((NQ, 2)), full((NQ, 2)),
            full((2 * IN, 2 * H)), full((1, 2 * H)),
            full((2 * H, H)), full((1, H)),
            full((2 * IN, 2 * H)), full((1, 2 * H)),
            full((2 * H, H)), full((1, H)),
        ],
        out_specs=pl.BlockSpec((B, H), lambda it, eid: (it, 0)),
        scratch_shapes=[pltpu.VMEM((B, 2 * H), jnp.bfloat16)],
    )
    y_sorted = pl.pallas_call(
        _mlp_body,
        grid_spec=grid_spec,
        out_shape=jax.ShapeDtypeStruct((NPAD, H), jnp.float32),
    )(eid, sig_sorted, emb_app_c, emb_tf_c, emb_app_l, emb_tf_l,
      W1cs, b1c2, W2cb, b2c2, W1ls, b1l2, W2lb, b2l2)

    return _make_row_gather(N, H, jnp.float32, chunks=[24] * 10 + [16])(y_sorted, row)
